# baseline (speedup 1.0000x reference)
import functools

import jax
import jax.numpy as jnp
from jax import lax
from jax.experimental import pallas as pl
from jax.experimental.pallas import tpu as pltpu
from jax.experimental.pallas import tpu_sc as plsc

B = 16384
D = 64
NNEG = 20
IPAD = 32
CB = 32
NSLAB = CB * NNEG // 128

_GDN = lax.GatherDimensionNumbers(
    offset_dims=(), collapsed_slice_dims=(0,), start_index_map=(0,))


def _lane_perm(x, idx):
    return lax.gather(x, idx[:, None], _GDN, slice_sizes=(1,),
                      mode=lax.GatherScatterMode.PROMISE_IN_BOUNDS)


@functools.cache
def _build_sc_scores(nc: int, ns: int):
    nw = nc * ns
    bpw = B // nw
    nchunk = bpw // CB
    mesh = plsc.VectorSubcoreMesh(core_axis_name="c", subcore_axis_name="s")

    def body(uidf_h, prow_h, ppar_h, nrow_h, npar_h, utab, itab, out,
             uidf, idx_p, par_p, idx_n, par_n,
             ubufs, p_rows, n_rows, obuf, sem):
        wid = lax.axis_index("s") * nc + lax.axis_index("c")
        lane = lax.iota(jnp.int32, 16)

        def scal(vec, lsel):
            return jnp.sum(jnp.where(lane == lsel, vec, 0.0)).astype(jnp.int32)

        @pl.loop(0, nchunk)
        def _chunk(ci):
            g = wid * nchunk + ci
            base = g * CB
            pltpu.sync_copy(uidf_h.at[pl.ds(base, CB)], uidf)
            pltpu.sync_copy(prow_h.at[pl.ds(base, CB)], idx_p)
            pltpu.sync_copy(ppar_h.at[pl.ds(base, CB)], par_p)
            for j in range(NSLAB):
                pltpu.sync_copy(nrow_h.at[g * NSLAB + j], idx_n.at[j])
                pltpu.sync_copy(npar_h.at[g * NSLAB + j],
                                par_n.at[pl.ds(j * 128, 128)])
            cps = [pltpu.async_copy(itab.at[idx_p], p_rows, sem)]
            for j in range(NSLAB):
                cps.append(pltpu.async_copy(
                    itab.at[idx_n.at[j]], n_rows.at[pl.ds(j * 128, 128)], sem))
            for b in range(CB):
                uv = uidf[pl.ds((b // 16) * 16, 16)]
                uid = scal(uv, b % 16)
                ublk = pl.multiple_of(uid & jnp.int32(~7), 8)
                cps.append(pltpu.async_copy(
                    utab.at[pl.ds(ublk, 8)], ubufs.at[b], sem))
            for cp in cps:
                cp.wait()

            @pl.loop(0, CB)
            def _row(b):
                b_lo = lax.bitwise_and(b, 15)
                b_hi = b - b_lo
                uid = scal(uidf[pl.ds(b_hi, 16)], b_lo)
                urow = uid & 7
                u = [ubufs[b, urow, pl.ds(k * 16, 16)] for k in range(4)]
                bsel = jnp.full((16,), b_lo, jnp.int32)

                def dot(ref, r, pmask):
                    lo = u[0] * ref[r, pl.ds(0, 16)]
                    hi = u[0] * ref[r, pl.ds(64, 16)]
                    for k in range(1, 4):
                        lo = lo + u[k] * ref[r, pl.ds(k * 16, 16)]
                        hi = hi + u[k] * ref[r, pl.ds(64 + k * 16, 16)]
                    acc = jnp.where(pmask, hi, lo)
                    for st in (8, 4, 2, 1):
                        acc = acc + _lane_perm(acc, lane ^ st)
                    return acc

                pm_p = _lane_perm(par_p[pl.ds(b_hi, 16)], bsel) == 1
                zero = jnp.zeros((16,), jnp.float32)
                r0 = jnp.where(lane == 0, dot(p_rows, b, pm_p), zero)
                r1 = zero
                pv1 = par_n[pl.ds(b * NNEG, 16)]
                pv2 = par_n[pl.ds(b * NNEG + 16, 16)]
                for n in range(NNEG):
                    col = n + 1
                    if n < 16:
                        pm = _lane_perm(pv1, jnp.full((16,), n, jnp.int32)) == 1
                    else:
                        pm = _lane_perm(pv2,
                                        jnp.full((16,), n - 16, jnp.int32)) == 1
                    total = dot(n_rows, b * NNEG + n, pm)
                    if col < 16:
                        r0 = jnp.where(lane == col, total, r0)
                    else:
                        r1 = jnp.where(lane == col - 16, total, r1)
                obuf[pl.ds(b * IPAD, 16)] = r0
                obuf[pl.ds(b * IPAD + 16, 16)] = r1

            pltpu.sync_copy(obuf, out.at[pl.ds(base * IPAD, CB * IPAD)])

    return pl.kernel(
        body,
        out_type=jax.ShapeDtypeStruct((B * IPAD,), jnp.float32),
        mesh=mesh,
        compiler_params=pltpu.CompilerParams(
            use_tc_tiling_on_sc=True, needs_layout_passes=False),
        scratch_types=[
            pltpu.VMEM((CB,), jnp.float32),
            pltpu.VMEM((CB,), jnp.int32),
            pltpu.VMEM((CB,), jnp.int32),
            pltpu.VMEM((NSLAB, 128), jnp.int32),
            pltpu.VMEM((CB * NNEG + 32,), jnp.int32),
            pltpu.VMEM((CB, 8, D), jnp.float32),
            pltpu.VMEM((CB, 128), jnp.float32),
            pltpu.VMEM((CB * NNEG, 128), jnp.float32),
            pltpu.VMEM((CB * IPAD,), jnp.float32),
            pltpu.SemaphoreType.DMA,
        ],
    )


def _loss_body(s_ref, o_ref):
    x = s_ref[...]
    col = lax.broadcasted_iota(jnp.int32, (B, IPAD), 1)
    pos = jnp.sum(jnp.where(col == 0, x, 0.0), axis=1, keepdims=True)
    lval = jnp.log(jax.nn.sigmoid(pos - x) + 1e-10)
    valid = (col >= 1) & (col <= NNEG)
    o_ref[0, 0] = -jnp.sum(jnp.where(valid, lval, 0.0)) * (1.0 / (B * NNEG))


_loss = pl.pallas_call(
    _loss_body,
    out_shape=jax.ShapeDtypeStruct((1, 1), jnp.float32),
    out_specs=pl.BlockSpec(memory_space=pltpu.SMEM))


def kernel(user_ids, pos_item_ids, neg_item_ids, user_table, item_table):
    info = plsc.get_sparse_core_info()
    sc_scores = _build_sc_scores(info.num_cores, info.num_subcores)
    itab = item_table.reshape(500000, 128)
    nflat = neg_item_ids.reshape(B * NNEG // 128, 128)
    flat = sc_scores(
        user_ids.astype(jnp.float32),
        pos_item_ids >> 1, pos_item_ids & 1,
        nflat >> 1, nflat & 1,
        user_table, itab)
    return _loss(flat.reshape(B, IPAD))[0, 0]


# speedup vs baseline: 1.1170x; 1.1170x over previous
import functools

import jax
import jax.numpy as jnp
from jax import lax
from jax.experimental import pallas as pl
from jax.experimental.pallas import tpu as pltpu
from jax.experimental.pallas import tpu_sc as plsc

B = 16384
D = 64
NNEG = 20
IPAD = 32
CB = 32
KV = D // 16
NSLAB = CB * NNEG // 128

_GDN = lax.GatherDimensionNumbers(
    offset_dims=(), collapsed_slice_dims=(0,), start_index_map=(0,))


def _lane_perm(x, idx):
    return lax.gather(x, idx[:, None], _GDN, slice_sizes=(1,),
                      mode=lax.GatherScatterMode.PROMISE_IN_BOUNDS)


@functools.cache
def _build_user_gather(nc: int, ns: int):
    nw = nc * ns
    bpw = B // nw
    nchunk = bpw // CB
    mesh = plsc.VectorSubcoreMesh(core_axis_name="c", subcore_axis_name="s")

    def body(uidf_h, utab, out, uidf, ubufs, obuf, sem):
        wid = lax.axis_index("s") * nc + lax.axis_index("c")
        lane = lax.iota(jnp.int32, 16)

        def scal(vec, lsel):
            return jnp.sum(jnp.where(lane == lsel, vec, 0.0)).astype(jnp.int32)

        @pl.loop(0, nchunk)
        def _chunk(ci):
            g = wid * nchunk + ci
            base = g * CB
            pltpu.sync_copy(uidf_h.at[pl.ds(base, CB)], uidf)
            cps = []
            for b in range(CB):
                uv = uidf[pl.ds((b // 16) * 16, 16)]
                uid = scal(uv, b % 16)
                ublk = pl.multiple_of(uid & jnp.int32(~7), 8)
                cps.append(pltpu.async_copy(
                    utab.at[pl.ds(ublk, 8)], ubufs.at[b], sem))
            for cp in cps:
                cp.wait()

            @pl.loop(0, CB)
            def _row(b):
                b_lo = lax.bitwise_and(b, 15)
                b_hi = b - b_lo
                uid = scal(uidf[pl.ds(b_hi, 16)], b_lo)
                urow = uid & 7
                for k in range(KV):
                    obuf[pl.ds(b * D + k * 16, 16)] = (
                        ubufs[b, urow, pl.ds(k * 16, 16)])

            pltpu.sync_copy(obuf, out.at[pl.ds(base * D, CB * D)])

    return pl.kernel(
        body,
        out_type=jax.ShapeDtypeStruct((B * D,), jnp.float32),
        mesh=mesh,
        compiler_params=pltpu.CompilerParams(
            use_tc_tiling_on_sc=True, needs_layout_passes=False),
        scratch_types=[
            pltpu.VMEM((CB,), jnp.float32),
            pltpu.VMEM((CB, 8, D), jnp.float32),
            pltpu.VMEM((CB * D,), jnp.float32),
            pltpu.SemaphoreType.DMA,
        ],
    )


@functools.cache
def _build_sc_scores(nc: int, ns: int):
    nw = nc * ns
    bpw = B // nw
    nchunk = bpw // CB
    mesh = plsc.VectorSubcoreMesh(core_axis_name="c", subcore_axis_name="s")

    def body(uemb, pid_h, nid_h, itab, out,
             idx_p, idx_n, u_rows, p_rows, n_rows, obuf, sem):
        wid = lax.axis_index("s") * nc + lax.axis_index("c")
        lane = lax.iota(jnp.int32, 16)

        @pl.loop(0, nchunk)
        def _chunk(ci):
            g = wid * nchunk + ci
            base = g * CB
            pltpu.sync_copy(pid_h.at[pl.ds(base, CB)], idx_p)
            pltpu.sync_copy(uemb.at[pl.ds(base * D, CB * D)], u_rows)
            for j in range(NSLAB):
                pltpu.sync_copy(nid_h.at[g * NSLAB + j], idx_n.at[j])
            cps = [pltpu.async_copy(itab.at[idx_p], p_rows, sem)]
            for j in range(NSLAB):
                cps.append(pltpu.async_copy(
                    itab.at[idx_n.at[j]], n_rows.at[pl.ds(j * 128, 128)], sem))
            for cp in cps:
                cp.wait()

            @pl.loop(0, CB)
            def _row(b):
                u = [u_rows[pl.ds(b * D + k * 16, 16)] for k in range(KV)]

                def dot(ref, r):
                    acc = u[0] * ref[r, pl.ds(0, 16)]
                    for k in range(1, KV):
                        acc = acc + u[k] * ref[r, pl.ds(k * 16, 16)]
                    for s in (8, 4, 2, 1):
                        acc = acc + _lane_perm(acc, lane ^ s)
                    return acc

                zero = jnp.zeros((16,), jnp.float32)
                r0 = jnp.where(lane == 0, dot(p_rows, b), zero)
                r1 = zero
                for n in range(NNEG):
                    col = n + 1
                    total = dot(n_rows, b * NNEG + n)
                    if col < 16:
                        r0 = jnp.where(lane == col, total, r0)
                    else:
                        r1 = jnp.where(lane == col - 16, total, r1)
                obuf[pl.ds(b * IPAD, 16)] = r0
                obuf[pl.ds(b * IPAD + 16, 16)] = r1

            pltpu.sync_copy(obuf, out.at[pl.ds(base * IPAD, CB * IPAD)])

    return pl.kernel(
        body,
        out_type=jax.ShapeDtypeStruct((B * IPAD,), jnp.float32),
        mesh=mesh,
        compiler_params=pltpu.CompilerParams(use_tc_tiling_on_sc=False),
        scratch_types=[
            pltpu.VMEM((CB,), jnp.int32),
            pltpu.VMEM((NSLAB, 128), jnp.int32),
            pltpu.VMEM((CB * D,), jnp.float32),
            pltpu.VMEM((CB, D), jnp.float32),
            pltpu.VMEM((CB * NNEG, D), jnp.float32),
            pltpu.VMEM((CB * IPAD,), jnp.float32),
            pltpu.SemaphoreType.DMA,
        ],
    )


def _loss_body(s_ref, o_ref):
    x = s_ref[...]
    col = lax.broadcasted_iota(jnp.int32, (B, IPAD), 1)
    pos = jnp.sum(jnp.where(col == 0, x, 0.0), axis=1, keepdims=True)
    lval = jnp.log(jax.nn.sigmoid(pos - x) + 1e-10)
    valid = (col >= 1) & (col <= NNEG)
    o_ref[0, 0] = -jnp.sum(jnp.where(valid, lval, 0.0)) * (1.0 / (B * NNEG))


_loss = pl.pallas_call(
    _loss_body,
    out_shape=jax.ShapeDtypeStruct((1, 1), jnp.float32),
    out_specs=pl.BlockSpec(memory_space=pltpu.SMEM))


def kernel(user_ids, pos_item_ids, neg_item_ids, user_table, item_table):
    info = plsc.get_sparse_core_info()
    user_gather = _build_user_gather(info.num_cores, info.num_subcores)
    sc_scores = _build_sc_scores(info.num_cores, info.num_subcores)
    uemb = user_gather(user_ids.astype(jnp.float32), user_table)
    nid = neg_item_ids.reshape(B * NNEG // 128, 128)
    flat = sc_scores(uemb, pos_item_ids, nid, item_table)
    return _loss(flat.reshape(B, IPAD))[0, 0]


# speedup vs baseline: 1.3798x; 1.2352x over previous
import functools

import jax
import jax.numpy as jnp
from jax import lax
from jax.experimental import pallas as pl
from jax.experimental.pallas import tpu as pltpu
from jax.experimental.pallas import tpu_sc as plsc

B = 16384
D = 64
NNEG = 20
IPAD = 32
CB = 32
KV = D // 16
NSLAB = CB * NNEG // 128

_GDN = lax.GatherDimensionNumbers(
    offset_dims=(), collapsed_slice_dims=(0,), start_index_map=(0,))


def _lane_perm(x, idx):
    return lax.gather(x, idx[:, None], _GDN, slice_sizes=(1,),
                      mode=lax.GatherScatterMode.PROMISE_IN_BOUNDS)


@functools.cache
def _build_user_gather(nc: int, ns: int):
    nw = nc * ns
    bpw = B // nw
    nchunk = bpw // CB
    mesh = plsc.VectorSubcoreMesh(core_axis_name="c", subcore_axis_name="s")

    def body(uidf_h, utab_t, out, uidf, ubufs, obuf, sem):
        wid = lax.axis_index("s") * nc + lax.axis_index("c")
        lane = lax.iota(jnp.int32, 16)

        def scal(vec, lsel):
            return jnp.sum(jnp.where(lane == lsel, vec, 0.0)).astype(jnp.int32)

        @pl.loop(0, nchunk)
        def _chunk(ci):
            g = wid * nchunk + ci
            base = g * CB
            pltpu.sync_copy(uidf_h.at[pl.ds(base, CB)], uidf)
            for sub in range(CB // 8):
                cps = []
                uids = []
                for i in range(8):
                    b = sub * 8 + i
                    uv = uidf[pl.ds((b // 16) * 16, 16)]
                    uid = scal(uv, b % 16)
                    uids.append(uid)
                    ublk = pl.multiple_of(uid & jnp.int32(~127), 128)
                    cps.append(pltpu.async_copy(
                        utab_t.at[:, pl.ds(ublk, 128)], ubufs.at[i], sem))
                for cp in cps:
                    cp.wait()
                for i in range(8):
                    b = sub * 8 + i
                    col = jnp.full((16,), uids[i] & 127, jnp.int32)
                    for k in range(KV):
                        rows16 = lane + k * 16
                        obuf[pl.ds(b * D + k * 16, 16)] = (
                            plsc.load_gather(ubufs.at[i], [rows16, col]))

            pltpu.sync_copy(obuf, out.at[pl.ds(base * D, CB * D)])

    return pl.kernel(
        body,
        out_type=jax.ShapeDtypeStruct((B * D,), jnp.float32),
        mesh=mesh,
        compiler_params=pltpu.CompilerParams(
            use_tc_tiling_on_sc=True, needs_layout_passes=False),
        scratch_types=[
            pltpu.VMEM((CB,), jnp.float32),
            pltpu.VMEM((8, D, 128), jnp.float32),
            pltpu.VMEM((CB * D,), jnp.float32),
            pltpu.SemaphoreType.DMA,
        ],
    )


@functools.cache
def _build_sc_scores(nc: int, ns: int):
    nw = nc * ns
    bpw = B // nw
    nchunk = bpw // CB
    mesh = plsc.VectorSubcoreMesh(core_axis_name="c", subcore_axis_name="s")

    def body(uemb, pid_h, nid_h, itab, out,
             idx_p, idx_n, u_rows, p_rows, n_rows, obuf, sem):
        wid = lax.axis_index("s") * nc + lax.axis_index("c")
        lane = lax.iota(jnp.int32, 16)

        @pl.loop(0, nchunk)
        def _chunk(ci):
            g = wid * nchunk + ci
            base = g * CB
            pltpu.sync_copy(pid_h.at[pl.ds(base, CB)], idx_p)
            pltpu.sync_copy(uemb.at[pl.ds(base * D, CB * D)], u_rows)
            for j in range(NSLAB):
                pltpu.sync_copy(nid_h.at[g * NSLAB + j], idx_n.at[j])
            cps = [pltpu.async_copy(itab.at[idx_p], p_rows, sem)]
            for j in range(NSLAB):
                cps.append(pltpu.async_copy(
                    itab.at[idx_n.at[j]], n_rows.at[pl.ds(j * 128, 128)], sem))
            for cp in cps:
                cp.wait()

            @pl.loop(0, CB)
            def _row(b):
                u = [u_rows[pl.ds(b * D + k * 16, 16)] for k in range(KV)]

                def dot(ref, r):
                    acc = u[0] * ref[r, pl.ds(0, 16)]
                    for k in range(1, KV):
                        acc = acc + u[k] * ref[r, pl.ds(k * 16, 16)]
                    for s in (8, 4, 2, 1):
                        acc = acc + _lane_perm(acc, lane ^ s)
                    return acc

                zero = jnp.zeros((16,), jnp.float32)
                r0 = jnp.where(lane == 0, dot(p_rows, b), zero)
                r1 = zero
                for n in range(NNEG):
                    col = n + 1
                    total = dot(n_rows, b * NNEG + n)
                    if col < 16:
                        r0 = jnp.where(lane == col, total, r0)
                    else:
                        r1 = jnp.where(lane == col - 16, total, r1)
                obuf[pl.ds(b * IPAD, 16)] = r0
                obuf[pl.ds(b * IPAD + 16, 16)] = r1

            pltpu.sync_copy(obuf, out.at[pl.ds(base * IPAD, CB * IPAD)])

    return pl.kernel(
        body,
        out_type=jax.ShapeDtypeStruct((B * IPAD,), jnp.float32),
        mesh=mesh,
        compiler_params=pltpu.CompilerParams(use_tc_tiling_on_sc=False),
        scratch_types=[
            pltpu.VMEM((CB,), jnp.int32),
            pltpu.VMEM((NSLAB, 128), jnp.int32),
            pltpu.VMEM((CB * D,), jnp.float32),
            pltpu.VMEM((CB, D), jnp.float32),
            pltpu.VMEM((CB * NNEG, D), jnp.float32),
            pltpu.VMEM((CB * IPAD,), jnp.float32),
            pltpu.SemaphoreType.DMA,
        ],
    )


def _loss_body(s_ref, o_ref):
    x = s_ref[...]
    col = lax.broadcasted_iota(jnp.int32, (B, IPAD), 1)
    pos = jnp.sum(jnp.where(col == 0, x, 0.0), axis=1, keepdims=True)
    lval = jnp.log(jax.nn.sigmoid(pos - x) + 1e-10)
    valid = (col >= 1) & (col <= NNEG)
    o_ref[0, 0] = -jnp.sum(jnp.where(valid, lval, 0.0)) * (1.0 / (B * NNEG))


_loss = pl.pallas_call(
    _loss_body,
    out_shape=jax.ShapeDtypeStruct((1, 1), jnp.float32),
    out_specs=pl.BlockSpec(memory_space=pltpu.SMEM))


def kernel(user_ids, pos_item_ids, neg_item_ids, user_table, item_table):
    info = plsc.get_sparse_core_info()
    user_gather = _build_user_gather(info.num_cores, info.num_subcores)
    sc_scores = _build_sc_scores(info.num_cores, info.num_subcores)
    uemb = user_gather(user_ids.astype(jnp.float32), user_table.T)
    nid = neg_item_ids.reshape(B * NNEG // 128, 128)
    flat = sc_scores(uemb, pos_item_ids, nid, item_table)
    return _loss(flat.reshape(B, IPAD))[0, 0]


# speedup vs baseline: 1.4523x; 1.0525x over previous
import functools

import jax
import jax.numpy as jnp
from jax import lax
from jax.experimental import pallas as pl
from jax.experimental.pallas import tpu as pltpu
from jax.experimental.pallas import tpu_sc as plsc

B = 16384
D = 64
NNEG = 20
IPAD = 32
CB = 32
KV = D // 16
NSLAB = CB * NNEG // 128

_GDN = lax.GatherDimensionNumbers(
    offset_dims=(), collapsed_slice_dims=(0,), start_index_map=(0,))


def _lane_perm(x, idx):
    return lax.gather(x, idx[:, None], _GDN, slice_sizes=(1,),
                      mode=lax.GatherScatterMode.PROMISE_IN_BOUNDS)


@functools.cache
def _build_user_gather(nc: int, ns: int):
    nw = nc * ns
    bpw = B // nw
    nchunk = bpw // CB
    mesh = plsc.VectorSubcoreMesh(core_axis_name="c", subcore_axis_name="s")

    def body(uidf_h, utab_t, out, uidf, ubufs, obuf, sem):
        wid = lax.axis_index("s") * nc + lax.axis_index("c")
        lane = lax.iota(jnp.int32, 16)

        def scal(vec, lsel):
            return jnp.sum(jnp.where(lane == lsel, vec, 0.0)).astype(jnp.int32)

        @pl.loop(0, nchunk)
        def _chunk(ci):
            g = wid * nchunk + ci
            base = g * CB
            pltpu.sync_copy(uidf_h.at[pl.ds(base, CB)], uidf)
            for sub in range(CB // 8):
                cps = []
                uids = []
                for i in range(8):
                    b = sub * 8 + i
                    uv = uidf[pl.ds((b // 16) * 16, 16)]
                    uid = scal(uv, b % 16)
                    uids.append(uid)
                    ublk = pl.multiple_of(uid & jnp.int32(~127), 128)
                    cps.append(pltpu.async_copy(
                        utab_t.at[:, pl.ds(ublk, 128)], ubufs.at[i], sem))
                for cp in cps:
                    cp.wait()
                for i in range(8):
                    b = sub * 8 + i
                    col = jnp.full((16,), uids[i] & 127, jnp.int32)
                    for k in range(KV):
                        rows16 = lane + k * 16
                        obuf[pl.ds(b * D + k * 16, 16)] = (
                            plsc.load_gather(ubufs.at[i], [rows16, col]))

            pltpu.sync_copy(obuf, out.at[pl.ds(base * D, CB * D)])

    return pl.kernel(
        body,
        out_type=jax.ShapeDtypeStruct((B * D,), jnp.float32),
        mesh=mesh,
        compiler_params=pltpu.CompilerParams(
            use_tc_tiling_on_sc=True, needs_layout_passes=False),
        scratch_types=[
            pltpu.VMEM((CB,), jnp.float32),
            pltpu.VMEM((8, D, 128), jnp.float32),
            pltpu.VMEM((CB * D,), jnp.float32),
            pltpu.SemaphoreType.DMA,
        ],
    )


@functools.cache
def _build_sc_scores(nc: int, ns: int):
    nw = nc * ns
    bpw = B // nw
    nchunk = bpw // CB
    mesh = plsc.VectorSubcoreMesh(core_axis_name="c", subcore_axis_name="s")

    def body(uemb, pid_h, nid_h, itab, out,
             idx_p, idx_n, u_rows, p_rows, n_rows, obuf, sem, sem2):
        wid = lax.axis_index("s") * nc + lax.axis_index("c")
        lane = lax.iota(jnp.int32, 16)

        @pl.loop(0, nchunk)
        def _chunk(ci):
            g = wid * nchunk + ci
            base = g * CB
            stage = [
                pltpu.async_copy(pid_h.at[pl.ds(base, CB)], idx_p, sem2),
                pltpu.async_copy(uemb.at[pl.ds(base * D, CB * D)], u_rows,
                                 sem2),
            ]
            for j in range(NSLAB):
                stage.append(pltpu.async_copy(
                    nid_h.at[g * NSLAB + j], idx_n.at[j], sem2))
            for cp in stage:
                cp.wait()
            cps = [pltpu.async_copy(itab.at[idx_p], p_rows, sem)]
            for j in range(NSLAB):
                cps.append(pltpu.async_copy(
                    itab.at[idx_n.at[j]], n_rows.at[pl.ds(j * 128, 128)], sem))
            for cp in cps:
                cp.wait()

            @pl.loop(0, CB)
            def _row(b):
                u = [u_rows[pl.ds(b * D + k * 16, 16)] for k in range(KV)]

                def dot(ref, r):
                    acc = u[0] * ref[r, pl.ds(0, 16)]
                    for k in range(1, KV):
                        acc = acc + u[k] * ref[r, pl.ds(k * 16, 16)]
                    for s in (8, 4, 2, 1):
                        acc = acc + _lane_perm(acc, lane ^ s)
                    return acc

                zero = jnp.zeros((16,), jnp.float32)
                r0 = jnp.where(lane == 0, dot(p_rows, b), zero)
                r1 = zero
                for n in range(NNEG):
                    col = n + 1
                    total = dot(n_rows, b * NNEG + n)
                    if col < 16:
                        r0 = jnp.where(lane == col, total, r0)
                    else:
                        r1 = jnp.where(lane == col - 16, total, r1)
                obuf[pl.ds(b * IPAD, 16)] = r0
                obuf[pl.ds(b * IPAD + 16, 16)] = r1

            pltpu.sync_copy(obuf, out.at[pl.ds(base * IPAD, CB * IPAD)])

    return pl.kernel(
        body,
        out_type=jax.ShapeDtypeStruct((B * IPAD,), jnp.float32),
        mesh=mesh,
        compiler_params=pltpu.CompilerParams(use_tc_tiling_on_sc=False),
        scratch_types=[
            pltpu.VMEM((CB,), jnp.int32),
            pltpu.VMEM((NSLAB, 128), jnp.int32),
            pltpu.VMEM((CB * D,), jnp.float32),
            pltpu.VMEM((CB, D), jnp.float32),
            pltpu.VMEM((CB * NNEG, D), jnp.float32),
            pltpu.VMEM((CB * IPAD,), jnp.float32),
            pltpu.SemaphoreType.DMA,
            pltpu.SemaphoreType.DMA,
        ],
    )


def _loss_body(s_ref, o_ref):
    x = s_ref[...]
    col = lax.broadcasted_iota(jnp.int32, (B, IPAD), 1)
    pos = jnp.sum(jnp.where(col == 0, x, 0.0), axis=1, keepdims=True)
    lval = jnp.log(jax.nn.sigmoid(pos - x) + 1e-10)
    valid = (col >= 1) & (col <= NNEG)
    o_ref[0, 0] = -jnp.sum(jnp.where(valid, lval, 0.0)) * (1.0 / (B * NNEG))


_loss = pl.pallas_call(
    _loss_body,
    out_shape=jax.ShapeDtypeStruct((1, 1), jnp.float32),
    out_specs=pl.BlockSpec(memory_space=pltpu.SMEM))


def kernel(user_ids, pos_item_ids, neg_item_ids, user_table, item_table):
    info = plsc.get_sparse_core_info()
    user_gather = _build_user_gather(info.num_cores, info.num_subcores)
    sc_scores = _build_sc_scores(info.num_cores, info.num_subcores)
    uemb = user_gather(user_ids.astype(jnp.float32), user_table.T)
    nid = neg_item_ids.reshape(B * NNEG // 128, 128)
    flat = sc_scores(uemb, pos_item_ids, nid, item_table)
    return _loss(flat.reshape(B, IPAD))[0, 0]
